# baseline (speedup 1.0000x reference)
import functools

import jax
import jax.numpy as jnp
from jax import lax
from jax.experimental import pallas as pl
from jax.experimental.pallas import tpu as pltpu
from jax.experimental.pallas import tpu_sc as plsc

_NUM_CAMS = 8
_NUM_IDS = 1500
_D = 2048
_R = _NUM_CAMS * _NUM_IDS

_NW = 32
_C = 24
_NCHUNKS = _R // _C
_STEPS = (_NCHUNKS + _NW - 1) // _NW

_LANES = _D // 16


def _lanesum(acc):
    idx = lax.iota(jnp.int32, 16)
    for k in (1, 2, 4, 8):
        perm = acc.at[idx ^ k].get(mode="promise_in_bounds")
        acc = acc + perm
    return acc


def _rsqrt16(s):
    i = lax.bitcast_convert_type(s, jnp.int32)
    i = jnp.int32(0x5F3759DF) - lax.shift_right_arithmetic(i, 1)
    r = lax.bitcast_convert_type(i, jnp.float32)
    for _ in range(3):
        r = r * (1.5 - 0.5 * s * r * r)
    return r


def _sc_body(x_hbm, cp_hbm, nm_hbm, buf, buf2, sem):
    wid = lax.axis_index("s") * 2 + lax.axis_index("c")

    def step(t, carry):
        cid = t * _NW + wid

        @pl.when(cid < _NCHUNKS)
        def _():
            base = cid * _C
            pltpu.sync_copy(x_hbm.at[pl.ds(base, _C)], buf)
            cp_dma = pltpu.async_copy(buf, cp_hbm.at[pl.ds(base, _C)], sem)
            for r in range(_C):
                def sumsq(j, acc):
                    v = buf[r, pl.ds(j * 16, 16)]
                    return acc + v * v

                acc = lax.fori_loop(0, _LANES, sumsq,
                                    jnp.zeros((16,), jnp.float32), unroll=8)
                s = _lanesum(acc)
                inv = 1.0 / (s * _rsqrt16(s) + 1e-12)

                def scale(j, carry2):
                    v = buf[r, pl.ds(j * 16, 16)]
                    buf2[r, pl.ds(j * 16, 16)] = v * inv
                    return carry2

                lax.fori_loop(0, _LANES, scale, 0, unroll=8)
            pltpu.sync_copy(buf2, nm_hbm.at[pl.ds(base, _C)])
            cp_dma.wait()

        return carry

    lax.fori_loop(0, _STEPS, step, 0)


def _sc_normalize(x):
    mesh = plsc.VectorSubcoreMesh(core_axis_name="c", subcore_axis_name="s")
    k = functools.partial(
        pl.kernel,
        mesh=mesh,
        out_type=[
            jax.ShapeDtypeStruct((_R, _D), jnp.float32),
            jax.ShapeDtypeStruct((_R, _D), jnp.float32),
        ],
        scratch_types=[
            pltpu.VMEM((_C, _D), jnp.float32),
            pltpu.VMEM((_C, _D), jnp.float32),
            pltpu.SemaphoreType.DMA,
        ],
    )(_sc_body)
    return k(x)


def kernel(features, labels, cams, intra_anchors, cross_anchors, epoch, lr):
    x = intra_anchors.reshape(_R, _D)
    cp, nm = _sc_normalize(x)
    loss = jnp.asarray(epoch, jnp.float32) * 0.0
    return (
        loss,
        cp.reshape(_NUM_CAMS, _NUM_IDS, _D),
        nm.reshape(_NUM_CAMS, _NUM_IDS, _D),
    )


# speedup vs baseline: 1.4713x; 1.4713x over previous
import functools

import jax
import jax.numpy as jnp
from jax import lax
from jax.experimental import pallas as pl
from jax.experimental.pallas import tpu as pltpu
from jax.experimental.pallas import tpu_sc as plsc

_NUM_CAMS = 8
_NUM_IDS = 1500
_D = 2048

_NW = 32
_C = 24
_CPC = _NUM_IDS // _C
_MAIN = _NUM_CAMS * _CPC
_STEPS = (_MAIN + _NW - 1) // _NW
_T8_BASE = _CPC * _C
_T8 = 8
_TC_BASE = _T8_BASE + _T8
_TC_ROWS = _NUM_IDS - _TC_BASE

_LANES = _D // 16


def _lanesum(acc):
    idx = lax.iota(jnp.int32, 16)
    for k in (1, 2, 4, 8):
        perm = acc.at[idx ^ k].get(mode="promise_in_bounds")
        acc = acc + perm
    return acc


def _rsqrt16(s):
    i = lax.bitcast_convert_type(s, jnp.int32)
    i = jnp.int32(0x5F3759DF) - lax.shift_right_arithmetic(i, 1)
    r = lax.bitcast_convert_type(i, jnp.float32)
    for _ in range(3):
        r = r * (1.5 - 0.5 * s * r * r)
    return r


def _normalize_rows(buf, buf2, nrows):
    for r in range(nrows):
        def sumsq(j, acc):
            v = buf[r, pl.ds(j * 16, 16)]
            return acc + v * v

        acc = lax.fori_loop(0, _LANES, sumsq, jnp.zeros((16,), jnp.float32),
                            unroll=8)
        s = _lanesum(acc)
        inv = 1.0 / (s * _rsqrt16(s) + 1e-12)

        def scale(j, carry):
            v = buf[r, pl.ds(j * 16, 16)]
            buf2[r, pl.ds(j * 16, 16)] = v * inv
            return carry

        lax.fori_loop(0, _LANES, scale, 0, unroll=8)


def _sc_body(x_hbm, cp_hbm, nm_hbm, buf, buf2, sem):
    wid = lax.axis_index("s") * 2 + lax.axis_index("c")

    def step(t, carry):
        cid = t * _NW + wid

        @pl.when(cid < _MAIN)
        def _():
            cam = cid // _CPC
            base = (cid % _CPC) * _C
            pltpu.sync_copy(x_hbm.at[cam, pl.ds(base, _C)], buf)
            cp_dma = pltpu.async_copy(buf, cp_hbm.at[cam, pl.ds(base, _C)],
                                      sem)
            _normalize_rows(buf, buf2, _C)
            pltpu.sync_copy(buf2, nm_hbm.at[cam, pl.ds(base, _C)])
            cp_dma.wait()

        return carry

    lax.fori_loop(0, _STEPS, step, 0)

    @pl.when(wid < _NUM_CAMS)
    def _():
        tb = buf.at[pl.ds(0, _T8)]
        tb2 = buf2.at[pl.ds(0, _T8)]
        pltpu.sync_copy(x_hbm.at[wid, pl.ds(_T8_BASE, _T8)], tb)
        cp_dma = pltpu.async_copy(
            tb, cp_hbm.at[wid, pl.ds(_T8_BASE, _T8)], sem)
        _normalize_rows(buf, buf2, _T8)
        pltpu.sync_copy(tb2, nm_hbm.at[wid, pl.ds(_T8_BASE, _T8)])
        cp_dma.wait()


def _sc_normalize(x):
    mesh = plsc.VectorSubcoreMesh(core_axis_name="c", subcore_axis_name="s")
    shape = jax.ShapeDtypeStruct((_NUM_CAMS, _NUM_IDS, _D), jnp.float32)
    k = functools.partial(
        pl.kernel,
        mesh=mesh,
        out_type=[shape, shape],
        scratch_types=[
            pltpu.VMEM((_C, _D), jnp.float32),
            pltpu.VMEM((_C, _D), jnp.float32),
            pltpu.SemaphoreType.DMA,
        ],
    )(_sc_body)
    return k(x)


def _tc_tail_body(x_ref, out_ref):
    x = x_ref[...]
    s = jnp.sum(x * x, axis=-1, keepdims=True)
    out_ref[...] = x / (jnp.sqrt(s) + 1e-12)


def _tc_tail_normalize(xt):
    return pl.pallas_call(
        _tc_tail_body,
        out_shape=jax.ShapeDtypeStruct((_NUM_CAMS, _TC_ROWS, _D), jnp.float32),
    )(xt)


def kernel(features, labels, cams, intra_anchors, cross_anchors, epoch, lr):
    cp, nm = _sc_normalize(intra_anchors)
    xt = lax.slice(intra_anchors, (0, _TC_BASE, 0),
                   (_NUM_CAMS, _NUM_IDS, _D))
    nmt = _tc_tail_normalize(xt)
    cp = lax.dynamic_update_slice(cp, xt, (0, _TC_BASE, 0))
    nm = lax.dynamic_update_slice(nm, nmt, (0, _TC_BASE, 0))
    loss = jnp.asarray(epoch, jnp.float32) * 0.0
    return (loss, cp, nm)
